# baseline (speedup 1.0000x reference)
import functools

import jax
import jax.numpy as jnp
from jax import lax
from jax.experimental import pallas as pl
from jax.experimental.pallas import tpu as pltpu
from jax.experimental.pallas import tpu_sc as plsc


def _build_sc_kernel(B, N_FIX, N_VAR, L, D, VOCAB):
    info = plsc.get_sparse_core_info()
    NC, NS = info.num_cores, info.num_subcores
    NW = NC * NS
    per_w = B // NW
    CB = 8
    n_chunks = per_w // CB
    NT = N_FIX + N_VAR
    NV = N_VAR * CB * L
    inv_l = float(1.0 / L)

    mesh = plsc.VectorSubcoreMesh(core_axis_name="c", subcore_axis_name="s")

    @functools.partial(
        pl.kernel,
        mesh=mesh,
        compiler_params=pltpu.CompilerParams(use_tc_tiling_on_sc=False),
        out_type=jax.ShapeDtypeStruct((B * NT, D), jnp.float32),
        scratch_types=[
            pltpu.VMEM((2 * CB * NT,), jnp.int32),
            pltpu.VMEM((2 * NV,), jnp.int32),
            pltpu.VMEM((2 * CB * NT, D), jnp.float32),
            pltpu.VMEM((2 * NV, D), jnp.float32),
            pltpu.SemaphoreType.DMA,
            pltpu.SemaphoreType.DMA,
            pltpu.SemaphoreType.DMA,
            pltpu.SemaphoreType.DMA,
            pltpu.SemaphoreType.DMA,
            pltpu.SemaphoreType.DMA,
        ],
    )
    def sc_kernel(wfix_hbm, wvar_hbm, mixidx_hbm, varidx_hbm, out_hbm,
                  mixidx_v, varidx_v, outbuf_v, varrows_v,
                  sem_g0, sem_g1, sem_out0, sem_out1, sem_idx0, sem_idx1):
        wid = lax.axis_index("s") * NC + lax.axis_index("c")
        sem_g = (sem_g0, sem_g1)
        sem_out = (sem_out0, sem_out1)
        sem_idx = (sem_idx0, sem_idx1)

        def mix_slice(c):
            return mixidx_hbm.at[pl.ds((wid * per_w + c * CB) * NT, CB * NT)]

        def var_slice(c):
            return varidx_hbm.at[pl.ds((wid * n_chunks + c) * NV, NV)]

        def out_slice(c):
            return out_hbm.at[pl.ds((wid * per_w + c * CB) * NT, CB * NT)]

        def mix_v(slot):
            return mixidx_v.at[pl.ds(slot * CB * NT, CB * NT)]

        def var_v(slot):
            return varidx_v.at[pl.ds(slot * NV, NV)]

        def outb_v(slot):
            return outbuf_v.at[pl.ds(slot * CB * NT, CB * NT)]

        def vrows_v(slot):
            return varrows_v.at[pl.ds(slot * NV, NV)]

        def issue_idx(c, slot):
            pltpu.async_copy(mix_slice(c), mix_v(slot), sem_idx[slot])
            pltpu.async_copy(var_slice(c), var_v(slot), sem_idx[slot])

        def wait_idx(slot):
            pltpu.make_async_copy(
                mix_slice(0), mix_v(slot), sem_idx[slot]).wait()
            pltpu.make_async_copy(
                var_slice(0), var_v(slot), sem_idx[slot]).wait()

        def issue_gathers(c, slot):
            pltpu.async_copy(wfix_hbm.at[mix_v(slot)],
                             outb_v(slot), sem_g[slot])
            pltpu.async_copy(wvar_hbm.at[var_v(slot)],
                             vrows_v(slot), sem_g[slot])

        def wait_gathers(slot):
            pltpu.make_async_copy(
                wfix_hbm.at[pl.ds(0, CB * NT)], outb_v(slot),
                sem_g[slot]).wait()
            pltpu.make_async_copy(
                wvar_hbm.at[pl.ds(0, NV)], vrows_v(slot),
                sem_g[slot]).wait()

        def wait_out(c, slot):
            pltpu.make_async_copy(
                outb_v(slot), out_slice(c), sem_out[slot]).wait()

        def pool(slot):
            vbase = slot * NV
            obase = slot * CB * NT

            def pool_body(b, carry):
                for v in range(N_VAR):
                    base = vbase + (v * CB + b) * L
                    r = obase + b * NT + N_FIX + v
                    for h in range(0, D, 16):
                        acc = varrows_v[base, pl.ds(h, 16)]
                        for l in range(1, L):
                            acc = acc + varrows_v[base + l, pl.ds(h, 16)]
                        outbuf_v[r, pl.ds(h, 16)] = acc * inv_l
                return carry

            lax.fori_loop(0, CB, pool_body, 0)

        def stage(c, slot, first=False, wait_idx_flag=True,
                  issue_next=True, issue_idx2=True):
            o = 1 - slot
            if issue_next:
                if not first:
                    wait_out(c, o)
                if wait_idx_flag:
                    wait_idx(o)
                issue_gathers(c + 1, o)
            wait_gathers(slot)
            if issue_idx2:
                issue_idx(c + 2, slot)
            pool(slot)
            pltpu.async_copy(outb_v(slot), out_slice(c), sem_out[slot])

        pltpu.sync_copy(mix_slice(0), mix_v(0))
        pltpu.sync_copy(var_slice(0), var_v(0))
        issue_gathers(0, 0)
        pltpu.sync_copy(mix_slice(1), mix_v(1))
        pltpu.sync_copy(var_slice(1), var_v(1))

        stage(0, 0, first=True, wait_idx_flag=False)

        def loop_body(i, carry):
            c = 2 * i + 1
            stage(c, 1)
            stage(c + 1, 0)
            return carry

        lax.fori_loop(0, (n_chunks - 4) // 2, loop_body, 0)

        stage(n_chunks - 3, 1)
        stage(n_chunks - 2, 0, issue_idx2=False)
        stage(n_chunks - 1, 1, issue_next=False, issue_idx2=False)

        wait_out(n_chunks - 2, 0)
        wait_out(n_chunks - 1, 1)

    return sc_kernel


def kernel(x_fix, x_varlen, W_fix, W_var):
    B, N_FIX = x_fix.shape
    _, N_VAR, L = x_varlen.shape
    VOCAB, D = W_fix.shape[1], W_fix.shape[2]
    NT = N_FIX + N_VAR
    CB = 8

    fix_idx = x_fix.astype(jnp.int32) + (
        jnp.arange(N_FIX, dtype=jnp.int32) * VOCAB)[None, :]
    mix_idx = jnp.concatenate(
        [fix_idx, jnp.zeros((B, N_VAR), jnp.int32)], axis=1)
    var_idx = (x_varlen.astype(jnp.int32) + (
        jnp.arange(N_VAR, dtype=jnp.int32) * VOCAB)[None, :, None])
    var_idx = var_idx.reshape(B // CB, CB, N_VAR, L).transpose(0, 2, 1, 3)

    sc_kernel = _build_sc_kernel(B, N_FIX, N_VAR, L, D, VOCAB)
    out = sc_kernel(
        W_fix.reshape(N_FIX * VOCAB, D),
        W_var.reshape(N_VAR * VOCAB, D),
        mix_idx.reshape(-1),
        var_idx.reshape(-1),
    )
    return out.reshape(B, NT * D)


# speedup vs baseline: 1.0208x; 1.0208x over previous
import functools

import jax
import jax.numpy as jnp
import numpy as np
from jax import lax
from jax.experimental import pallas as pl
from jax.experimental.pallas import tpu as pltpu
from jax.experimental.pallas import tpu_sc as plsc


def _build_sc_kernel(B, N_FIX, N_VAR, L, D, VOCAB):
    info = plsc.get_sparse_core_info()
    NC, NS = info.num_cores, info.num_subcores
    NW = NC * NS
    per_w = B // NW
    CB = 8
    n_chunks = per_w // CB
    NT = N_FIX + N_VAR
    NF = CB * N_FIX
    NM = CB * NT
    NV = N_VAR * CB * L
    inv_l = float(1.0 / L)

    mesh = plsc.VectorSubcoreMesh(core_axis_name="c", subcore_axis_name="s")

    @functools.partial(
        pl.kernel,
        mesh=mesh,
        compiler_params=pltpu.CompilerParams(use_tc_tiling_on_sc=False),
        out_type=jax.ShapeDtypeStruct((B * NT, D), jnp.float32),
        scratch_types=[
            pltpu.VMEM((2 * NF,), jnp.int32),
            pltpu.VMEM((2 * NM,), jnp.int32),
            pltpu.VMEM((2 * NV,), jnp.int32),
            pltpu.VMEM((NM,), jnp.int32),
            pltpu.VMEM((NV,), jnp.int32),
            pltpu.VMEM((2 * NM, D), jnp.float32),
            pltpu.VMEM((2 * NV, D), jnp.float32),
            pltpu.SemaphoreType.DMA,
            pltpu.SemaphoreType.DMA,
            pltpu.SemaphoreType.DMA,
            pltpu.SemaphoreType.DMA,
            pltpu.SemaphoreType.DMA,
            pltpu.SemaphoreType.DMA,
        ],
    )
    def sc_kernel(wfix_hbm, wvar_hbm, fixraw_hbm, varraw_hbm,
                  mixoff_hbm, varoff_hbm, out_hbm,
                  fixraw_v, mixidx_v, varidx_v, mixoff_v, varoff_v,
                  outbuf_v, varrows_v,
                  sem_g0, sem_g1, sem_out0, sem_out1, sem_idx0, sem_idx1):
        wid = lax.axis_index("s") * NC + lax.axis_index("c")
        sem_g = (sem_g0, sem_g1)
        sem_out = (sem_out0, sem_out1)
        sem_idx = (sem_idx0, sem_idx1)
        zero16 = jnp.zeros((16,), jnp.int32)

        def fix_slice(c):
            return fixraw_hbm.at[pl.ds((wid * per_w + c * CB) * N_FIX, NF)]

        def var_slice(c):
            return varraw_hbm.at[pl.ds((wid * per_w + c * CB) * N_VAR * L, NV)]

        def out_slice(c):
            return out_hbm.at[pl.ds((wid * per_w + c * CB) * NT, NM)]

        def fix_v(slot):
            return fixraw_v.at[pl.ds(slot * NF, NF)]

        def mix_v(slot):
            return mixidx_v.at[pl.ds(slot * NM, NM)]

        def var_v(slot):
            return varidx_v.at[pl.ds(slot * NV, NV)]

        def outb_v(slot):
            return outbuf_v.at[pl.ds(slot * NM, NM)]

        def vrows_v(slot):
            return varrows_v.at[pl.ds(slot * NV, NV)]

        def issue_idx(c, slot):
            pltpu.async_copy(fix_slice(c), fix_v(slot), sem_idx[slot])
            pltpu.async_copy(var_slice(c), var_v(slot), sem_idx[slot])

        def wait_idx(slot):
            pltpu.make_async_copy(
                fix_slice(0), fix_v(slot), sem_idx[slot]).wait()
            pltpu.make_async_copy(
                var_slice(0), var_v(slot), sem_idx[slot]).wait()

        def build_idx(slot):
            fbase = slot * NF
            mbase = slot * NM
            for b in range(CB):
                src = fbase + b * N_FIX
                dst = mbase + b * NT
                mixidx_v[pl.ds(dst, 16)] = fixraw_v[pl.ds(src, 16)]
                mixidx_v[pl.ds(dst + N_FIX - 16, 16)] = (
                    fixraw_v[pl.ds(src + N_FIX - 16, 16)])
            for k in range(NM // 16):
                p = mbase + k * 16
                mixidx_v[pl.ds(p, 16)] = (
                    mixidx_v[pl.ds(p, 16)] + mixoff_v[pl.ds(k * 16, 16)])
            vbase = slot * NV
            for k in range(NV // 16):
                p = vbase + k * 16
                varidx_v[pl.ds(p, 16)] = (
                    varidx_v[pl.ds(p, 16)] + varoff_v[pl.ds(k * 16, 16)])

        def issue_gathers(c, slot):
            pltpu.async_copy(wfix_hbm.at[mix_v(slot)],
                             outb_v(slot), sem_g[slot])
            pltpu.async_copy(wvar_hbm.at[var_v(slot)],
                             vrows_v(slot), sem_g[slot])

        def wait_gathers(slot):
            pltpu.make_async_copy(
                wfix_hbm.at[pl.ds(0, NM)], outb_v(slot),
                sem_g[slot]).wait()
            pltpu.make_async_copy(
                wvar_hbm.at[pl.ds(0, NV)], vrows_v(slot),
                sem_g[slot]).wait()

        def wait_out(c, slot):
            pltpu.make_async_copy(
                outb_v(slot), out_slice(c), sem_out[slot]).wait()

        def pool(slot):
            vbase = slot * NV
            obase = slot * NM

            def pool_body(b, carry):
                for v in range(N_VAR):
                    base = vbase + (b * N_VAR + v) * L
                    r = obase + b * NT + N_FIX + v
                    for h in range(0, D, 16):
                        acc = varrows_v[base, pl.ds(h, 16)]
                        for l in range(1, L):
                            acc = acc + varrows_v[base + l, pl.ds(h, 16)]
                        outbuf_v[r, pl.ds(h, 16)] = acc * inv_l
                return carry

            lax.fori_loop(0, CB, pool_body, 0)

        def stage(c, slot, first=False, wait_idx_flag=True,
                  issue_next=True, issue_idx2=True):
            o = 1 - slot
            if issue_next:
                if not first:
                    wait_out(c, o)
                if wait_idx_flag:
                    wait_idx(o)
                build_idx(o)
                issue_gathers(c + 1, o)
            wait_gathers(slot)
            if issue_idx2:
                issue_idx(c + 2, slot)
            pool(slot)
            pltpu.async_copy(outb_v(slot), out_slice(c), sem_out[slot])

        for k in range(2 * NM // 16):
            mixidx_v[pl.ds(k * 16, 16)] = zero16
        pltpu.sync_copy(mixoff_hbm, mixoff_v)
        pltpu.sync_copy(varoff_hbm, varoff_v)
        pltpu.sync_copy(fix_slice(0), fix_v(0))
        pltpu.sync_copy(var_slice(0), var_v(0))
        build_idx(0)
        issue_gathers(0, 0)
        pltpu.sync_copy(fix_slice(1), fix_v(1))
        pltpu.sync_copy(var_slice(1), var_v(1))

        stage(0, 0, first=True, wait_idx_flag=False)

        def loop_body(i, carry):
            c = 2 * i + 1
            stage(c, 1)
            stage(c + 1, 0)
            return carry

        lax.fori_loop(0, (n_chunks - 4) // 2, loop_body, 0)

        stage(n_chunks - 3, 1)
        stage(n_chunks - 2, 0, issue_idx2=False)
        stage(n_chunks - 1, 1, issue_next=False, issue_idx2=False)

        wait_out(n_chunks - 2, 0)
        wait_out(n_chunks - 1, 1)

    return sc_kernel


def kernel(x_fix, x_varlen, W_fix, W_var):
    B, N_FIX = x_fix.shape
    _, N_VAR, L = x_varlen.shape
    VOCAB, D = W_fix.shape[1], W_fix.shape[2]
    NT = N_FIX + N_VAR
    CB = 8

    mixoff = np.where(
        np.arange(NT, dtype=np.int32) < N_FIX,
        np.arange(NT, dtype=np.int32) * VOCAB, 0).astype(np.int32)
    mixoff = np.tile(mixoff, CB)
    varoff = np.repeat(
        np.arange(N_VAR, dtype=np.int32) * VOCAB, L)
    varoff = np.tile(varoff, CB)

    sc_kernel = _build_sc_kernel(B, N_FIX, N_VAR, L, D, VOCAB)
    out = sc_kernel(
        W_fix.reshape(N_FIX * VOCAB, D),
        W_var.reshape(N_VAR * VOCAB, D),
        x_fix.astype(jnp.int32).reshape(-1),
        x_varlen.astype(jnp.int32).reshape(-1),
        jnp.asarray(mixoff),
        jnp.asarray(varoff),
    )
    return out.reshape(B, NT * D)
